# baseline (device time: 17813 ns/iter reference)
import jax
import jax.numpy as jnp
from jax import lax
from jax.experimental import pallas as pl
from jax.experimental.pallas import tpu as pltpu

N_CHUNKS = 4


def kernel(x):
    m, n = x.shape
    half = n // 2
    rows = m // N_CHUNKS

    def body(x_ref, out_ref, stage_ref, send_sems, recv_sems):
        my_x = lax.axis_index("x")
        my_y = lax.axis_index("y")
        my_z = lax.axis_index("z")
        other = 1 - my_x

        barrier_sem = pltpu.get_barrier_semaphore()
        pl.semaphore_signal(
            barrier_sem, inc=1,
            device_id=(other, my_y, my_z),
            device_id_type=pl.DeviceIdType.MESH,
        )
        pl.semaphore_wait(barrier_sem, 1)

        rdmas = []
        for k in range(N_CHUNKS):
            stage_ref[pl.ds(k * rows, rows), :] = x_ref[
                pl.ds(k * rows, rows), pl.ds(other * half, half)
            ]
            rdma = pltpu.make_async_remote_copy(
                src_ref=stage_ref.at[pl.ds(k * rows, rows), :],
                dst_ref=out_ref.at[pl.ds(my_x * m + k * rows, rows), :],
                send_sem=send_sems.at[k],
                recv_sem=recv_sems.at[k],
                device_id=(other, my_y, my_z),
                device_id_type=pl.DeviceIdType.MESH,
            )
            rdma.start()
            rdmas.append(rdma)

        out_ref[pl.ds(my_x * m, m), :] = x_ref[:, pl.ds(my_x * half, half)]

        for rdma in rdmas:
            rdma.wait()

    return pl.pallas_call(
        body,
        out_shape=jax.ShapeDtypeStruct((2 * m, half), x.dtype),
        in_specs=[pl.BlockSpec(memory_space=pltpu.VMEM)],
        out_specs=pl.BlockSpec(memory_space=pltpu.VMEM),
        scratch_shapes=[
            pltpu.VMEM((m, half), x.dtype),
            pltpu.SemaphoreType.DMA((N_CHUNKS,)),
            pltpu.SemaphoreType.DMA((N_CHUNKS,)),
        ],
        compiler_params=pltpu.CompilerParams(collective_id=0),
    )(x)


# device time: 17735 ns/iter; 1.0044x vs baseline; 1.0044x over previous
import jax
import jax.numpy as jnp
from jax import lax
from jax.experimental import pallas as pl
from jax.experimental.pallas import tpu as pltpu


def kernel(x):
    m, n = x.shape
    half = n // 2

    def body(x_ref, out_ref, send_sem, recv_sem):
        my_x = lax.axis_index("x")
        my_y = lax.axis_index("y")
        my_z = lax.axis_index("z")
        other = 1 - my_x

        barrier_sem = pltpu.get_barrier_semaphore()
        pl.semaphore_signal(
            barrier_sem, inc=1,
            device_id=(other, my_y, my_z),
            device_id_type=pl.DeviceIdType.MESH,
        )
        pl.semaphore_wait(barrier_sem, 1)

        rdma = pltpu.make_async_remote_copy(
            src_ref=x_ref.at[:, pl.ds(other * half, half)],
            dst_ref=out_ref.at[pl.ds(my_x * m, m), :],
            send_sem=send_sem,
            recv_sem=recv_sem,
            device_id=(other, my_y, my_z),
            device_id_type=pl.DeviceIdType.MESH,
        )
        rdma.start()

        out_ref[pl.ds(my_x * m, m), :] = x_ref[:, pl.ds(my_x * half, half)]

        rdma.wait()

    return pl.pallas_call(
        body,
        out_shape=jax.ShapeDtypeStruct((2 * m, half), x.dtype),
        in_specs=[pl.BlockSpec(memory_space=pltpu.VMEM)],
        out_specs=pl.BlockSpec(memory_space=pltpu.VMEM),
        scratch_shapes=[
            pltpu.SemaphoreType.DMA,
            pltpu.SemaphoreType.DMA,
        ],
        compiler_params=pltpu.CompilerParams(collective_id=0),
    )(x)
